# baseline (speedup 1.0000x reference)
import dataclasses
import functools

import jax
import jax.numpy as jnp
from jax import lax
from jax.experimental import pallas as pl
from jax.experimental.pallas import tpu as pltpu
from jax.experimental.pallas import tpu_sc as plsc

N = 100000
D = 128
S = 1024
NC = 2
NS = 16
NW = NC * NS
BLK = 128
NFULL = N // BLK
TAIL = N - NFULL * BLK
KPW = NFULL // NW
REM = NFULL % NW
KMAX = KPW + 1
SEG_PER_TILE = S // NS
L = 16

_mesh = plsc.VectorSubcoreMesh(core_axis_name="c", subcore_axis_name="s")

_cp = pltpu.CompilerParams()
if "needs_layout_passes" in pltpu.CompilerParams.__dataclass_fields__:
    _cp = dataclasses.replace(_cp, needs_layout_passes=False)


@functools.partial(
    pl.kernel,
    compiler_params=_cp,
    out_type=(
        jax.ShapeDtypeStruct((NC, S, D), jnp.float32),
        jax.ShapeDtypeStruct((NW, S), jnp.float32),
    ),
    mesh=_mesh,
    scratch_types=[
        pltpu.VMEM((1, BLK), jnp.int32),
        pltpu.VMEM((1, BLK), jnp.int32),
        pltpu.VMEM((1, BLK), jnp.int32),
        pltpu.VMEM((BLK, D), jnp.float32),
        pltpu.VMEM((BLK, D), jnp.float32),
        pltpu.VMEM((BLK, D), jnp.float32),
        pltpu.VMEM((1, TAIL), jnp.int32),
        pltpu.VMEM((TAIL, D), jnp.float32),
        pltpu.VMEM((S,), jnp.float32),
        pltpu.VMEM((SEG_PER_TILE, D), jnp.float32),
        pltpu.VMEM_SHARED((S, D), jnp.float32),
        pltpu.SemaphoreType.DMA,
        pltpu.SemaphoreType.DMA,
        pltpu.SemaphoreType.DMA,
        pltpu.SemaphoreType.DMA,
        pltpu.SemaphoreType.DMA,
        pltpu.SemaphoreType.DMA,
        pltpu.SemaphoreType.DMA,
        pltpu.SemaphoreType.DMA,
        pltpu.SemaphoreType.DMA,
    ],
)
def _segsum_sc(h_hbm, ids_hbm, psum_hbm, pcnt_hbm,
               id0_v, id1_v, id2_v, rows0_v, rows1_v, rows2_v,
               idt_v, rowt_v, hist_v, zsum_v,
               sums_sh, si0, si1, si2, sr0, sr1, sr2, ss0, ss1, ss2):
    cid = lax.axis_index("c")
    sid = lax.axis_index("s")
    wid = cid * NS + sid
    start_w = wid * NFULL // NW
    cnt_w = (wid + 1) * NFULL // NW - start_w

    zero16 = jnp.zeros((L,), jnp.float32)
    one16 = jnp.ones((L,), jnp.float32)

    NB = 3
    ids_v = (id0_v, id1_v, id2_v)
    rows_v = (rows0_v, rows1_v, rows2_v)
    sem_id = (si0, si1, si2)
    sem_in = (sr0, sr1, sr2)
    sem_sc = (ss0, ss1, ss2)

    def _id_desc(k):
        return pltpu.make_async_copy(
            ids_hbm.at[pl.ds((start_w + k) * BLK, BLK)], ids_v[k % NB].at[0],
            sem_id[k % NB])

    def _in_desc(k):
        return pltpu.make_async_copy(
            h_hbm.at[pl.ds((start_w + k) * BLK, BLK)], rows_v[k % NB],
            sem_in[k % NB])

    def _sc_desc(k):
        return pltpu.make_async_copy(
            rows_v[k % NB], sums_sh.at[ids_v[k % NB].at[0]], sem_sc[k % NB])

    def start_in(k):
        @pl.when(k < cnt_w)
        def _():
            _id_desc(k).start()
            _in_desc(k).start()

    def wait_sc(k):
        @pl.when(k < cnt_w)
        def _():
            _sc_desc(k).wait()

    start_in(0)
    start_in(1)

    @pl.loop(0, S, step=L)
    def _(i):
        hist_v[pl.ds(i, L)] = zero16

    @pl.loop(0, SEG_PER_TILE)
    def _(i):
        @pl.loop(0, D, step=L)
        def _(j):
            zsum_v[i, pl.ds(j, L)] = zero16

    pltpu.sync_copy(zsum_v, sums_sh.at[pl.ds(sid * SEG_PER_TILE, SEG_PER_TILE)])

    plsc.subcore_barrier()

    for k in range(KMAX):
        if k + 2 < KMAX:
            if k >= 1:
                wait_sc(k - 1)
            start_in(k + 2)

        @pl.when(k < cnt_w)
        def _(k=k):
            _id_desc(k).wait()
            _in_desc(k).wait()
            pltpu.async_copy(rows_v[k % NB], sums_sh.at[ids_v[k % NB].at[0]],
                             sem_sc[k % NB], add=True)

            @pl.loop(0, BLK, step=L)
            def _(l):
                idx = ids_v[k % NB][0, pl.ds(l, L)]
                plsc.addupdate_scatter(hist_v, [idx], one16)
    wait_sc(KMAX - 3)
    wait_sc(KMAX - 2)
    wait_sc(KMAX - 1)

    @pl.when(wid == NW - 1)
    def _():
        base = NFULL * BLK
        pltpu.sync_copy(ids_hbm.at[pl.ds(base, TAIL)], idt_v.at[0])
        pltpu.sync_copy(h_hbm.at[pl.ds(base, TAIL)], rowt_v)
        pltpu.sync_copy(rowt_v, sums_sh.at[idt_v.at[0]], add=True)

        @pl.loop(0, TAIL, step=L)
        def _(l):
            idx = idt_v[0, pl.ds(l, L)]
            plsc.addupdate_scatter(hist_v, [idx], one16)

    plsc.subcore_barrier()

    lo = sid * SEG_PER_TILE
    pltpu.sync_copy(sums_sh.at[pl.ds(lo, SEG_PER_TILE)],
                    psum_hbm.at[cid, pl.ds(lo, SEG_PER_TILE)])
    pltpu.sync_copy(hist_v, pcnt_hbm.at[wid])


def _combine_tc(psum_ref, pcnt_ref, out_ref):
    sums = psum_ref[0] + psum_ref[1]
    cnt = jnp.maximum(jnp.sum(pcnt_ref[...], axis=0), 1.0)
    out_ref[...] = sums / cnt[:, None]


@jax.jit
def kernel(h, segment_ids):
    ids = segment_ids.astype(jnp.int32)
    psum, pcnt = _segsum_sc(h, ids)
    return pl.pallas_call(
        _combine_tc,
        out_shape=jax.ShapeDtypeStruct((S, D), jnp.float32),
    )(psum, pcnt)


# speedup vs baseline: 1.0320x; 1.0320x over previous
import dataclasses
import functools

import jax
import jax.numpy as jnp
from jax import lax
from jax.experimental import pallas as pl
from jax.experimental.pallas import tpu as pltpu
from jax.experimental.pallas import tpu_sc as plsc

N = 100000
D = 128
S = 1024
NC = 2
NS = 16
NW = NC * NS
BLK = 128
NFULL = N // BLK
TAIL = N - NFULL * BLK
KPW = NFULL // NW
REM = NFULL % NW
KMAX = KPW + 1
SEG_PER_TILE = S // NS
L = 16

_mesh = plsc.VectorSubcoreMesh(core_axis_name="c", subcore_axis_name="s")

_cp = pltpu.CompilerParams()
if "needs_layout_passes" in pltpu.CompilerParams.__dataclass_fields__:
    _cp = dataclasses.replace(_cp, needs_layout_passes=False)


@functools.partial(
    pl.kernel,
    compiler_params=_cp,
    out_type=(
        jax.ShapeDtypeStruct((NC, S, D), jnp.float32),
        jax.ShapeDtypeStruct((NW, S), jnp.float32),
    ),
    mesh=_mesh,
    scratch_types=[
        pltpu.VMEM((1, BLK), jnp.int32),
        pltpu.VMEM((1, BLK), jnp.int32),
        pltpu.VMEM((1, BLK), jnp.int32),
        pltpu.VMEM((BLK, D), jnp.float32),
        pltpu.VMEM((BLK, D), jnp.float32),
        pltpu.VMEM((BLK, D), jnp.float32),
        pltpu.VMEM((1, TAIL), jnp.int32),
        pltpu.VMEM((TAIL, D), jnp.float32),
        pltpu.VMEM((S,), jnp.float32),
        pltpu.VMEM((SEG_PER_TILE, D), jnp.float32),
        pltpu.VMEM_SHARED((S, D), jnp.float32),
        pltpu.SemaphoreType.DMA,
        pltpu.SemaphoreType.DMA,
        pltpu.SemaphoreType.DMA,
        pltpu.SemaphoreType.DMA,
        pltpu.SemaphoreType.DMA,
        pltpu.SemaphoreType.DMA,
        pltpu.SemaphoreType.DMA,
        pltpu.SemaphoreType.DMA,
        pltpu.SemaphoreType.DMA,
    ],
)
def _segsum_sc(h_hbm, ids_hbm, psum_hbm, pcnt_hbm,
               id0_v, id1_v, id2_v, rows0_v, rows1_v, rows2_v,
               idt_v, rowt_v, hist_v, zsum_v,
               sums_sh, si0, si1, si2, sr0, sr1, sr2, ss0, ss1, ss2):
    cid = lax.axis_index("c")
    sid = lax.axis_index("s")
    wid = cid * NS + sid
    start_w = wid * KPW + jnp.minimum(wid, REM)
    cnt_w = jnp.where(wid < REM, KPW + 1, KPW)

    zero16 = jnp.zeros((L,), jnp.float32)
    one16 = jnp.ones((L,), jnp.float32)

    NB = 3
    ids_v = (id0_v, id1_v, id2_v)
    rows_v = (rows0_v, rows1_v, rows2_v)
    sem_id = (si0, si1, si2)
    sem_in = (sr0, sr1, sr2)
    sem_sc = (ss0, ss1, ss2)

    def _id_desc(k):
        return pltpu.make_async_copy(
            ids_hbm.at[pl.ds((start_w + k) * BLK, BLK)], ids_v[k % NB].at[0],
            sem_id[k % NB])

    def _in_desc(k):
        return pltpu.make_async_copy(
            h_hbm.at[pl.ds((start_w + k) * BLK, BLK)], rows_v[k % NB],
            sem_in[k % NB])

    def _sc_desc(k):
        return pltpu.make_async_copy(
            rows_v[k % NB], sums_sh.at[ids_v[k % NB].at[0]], sem_sc[k % NB])

    def start_in(k):
        @pl.when(k < cnt_w)
        def _():
            _id_desc(k).start()
            _in_desc(k).start()

    def wait_sc(k):
        @pl.when(k < cnt_w)
        def _():
            _sc_desc(k).wait()

    start_in(0)
    start_in(1)

    @pl.loop(0, S, step=L)
    def _(i):
        hist_v[pl.ds(i, L)] = zero16

    @pl.loop(0, SEG_PER_TILE)
    def _(i):
        @pl.loop(0, D, step=L)
        def _(j):
            zsum_v[i, pl.ds(j, L)] = zero16

    pltpu.sync_copy(zsum_v, sums_sh.at[pl.ds(sid * SEG_PER_TILE, SEG_PER_TILE)])

    plsc.subcore_barrier()

    for k in range(KMAX):
        if k + 2 < KMAX:
            if k >= 1:
                wait_sc(k - 1)
            start_in(k + 2)

        @pl.when(k < cnt_w)
        def _(k=k):
            _id_desc(k).wait()
            _in_desc(k).wait()
            pltpu.async_copy(rows_v[k % NB], sums_sh.at[ids_v[k % NB].at[0]],
                             sem_sc[k % NB], add=True)

            @pl.loop(0, BLK, step=L)
            def _(l):
                idx = ids_v[k % NB][0, pl.ds(l, L)]
                plsc.addupdate_scatter(hist_v, [idx], one16)
    wait_sc(KMAX - 3)
    wait_sc(KMAX - 2)
    wait_sc(KMAX - 1)

    @pl.when(wid == NW - 1)
    def _():
        base = NFULL * BLK
        pltpu.sync_copy(ids_hbm.at[pl.ds(base, TAIL)], idt_v.at[0])
        pltpu.sync_copy(h_hbm.at[pl.ds(base, TAIL)], rowt_v)
        pltpu.sync_copy(rowt_v, sums_sh.at[idt_v.at[0]], add=True)

        @pl.loop(0, TAIL, step=L)
        def _(l):
            idx = idt_v[0, pl.ds(l, L)]
            plsc.addupdate_scatter(hist_v, [idx], one16)

    plsc.subcore_barrier()

    lo = sid * SEG_PER_TILE
    pltpu.sync_copy(sums_sh.at[pl.ds(lo, SEG_PER_TILE)],
                    psum_hbm.at[cid, pl.ds(lo, SEG_PER_TILE)])
    pltpu.sync_copy(hist_v, pcnt_hbm.at[wid])


def _combine_tc(psum_ref, pcnt_ref, out_ref):
    sums = psum_ref[0] + psum_ref[1]
    cnt = jnp.maximum(jnp.sum(pcnt_ref[...], axis=0), 1.0)
    out_ref[...] = sums / cnt[:, None]


@jax.jit
def kernel(h, segment_ids):
    ids = segment_ids.astype(jnp.int32)
    psum, pcnt = _segsum_sc(h, ids)
    return pl.pallas_call(
        _combine_tc,
        out_shape=jax.ShapeDtypeStruct((S, D), jnp.float32),
    )(psum, pcnt)


# speedup vs baseline: 1.0410x; 1.0087x over previous
import dataclasses
import functools

import jax
import jax.numpy as jnp
from jax import lax
from jax.experimental import pallas as pl
from jax.experimental.pallas import tpu as pltpu
from jax.experimental.pallas import tpu_sc as plsc

N = 100000
D = 128
S = 1024
NC = 2
NS = 16
NW = NC * NS
BLK = 128
NFULL = N // BLK
TAIL = N - NFULL * BLK
KPW = NFULL // NW
REM = NFULL % NW
KMAX = KPW + 1
SEG_PER_TILE = S // NS
L = 16

_mesh = plsc.VectorSubcoreMesh(core_axis_name="c", subcore_axis_name="s")

_cp = pltpu.CompilerParams()
if "needs_layout_passes" in pltpu.CompilerParams.__dataclass_fields__:
    _cp = dataclasses.replace(_cp, needs_layout_passes=False)


@functools.partial(
    pl.kernel,
    compiler_params=_cp,
    out_type=(
        jax.ShapeDtypeStruct((NC, S, D), jnp.float32),
        jax.ShapeDtypeStruct((NW, S), jnp.float32),
    ),
    mesh=_mesh,
    scratch_types=[
        pltpu.VMEM((1, BLK), jnp.int32),
        pltpu.VMEM((1, BLK), jnp.int32),
        pltpu.VMEM((1, BLK), jnp.int32),
        pltpu.VMEM((1, BLK), jnp.int32),
        pltpu.VMEM((BLK, D), jnp.float32),
        pltpu.VMEM((BLK, D), jnp.float32),
        pltpu.VMEM((BLK, D), jnp.float32),
        pltpu.VMEM((BLK, D), jnp.float32),
        pltpu.VMEM((1, TAIL), jnp.int32),
        pltpu.VMEM((TAIL, D), jnp.float32),
        pltpu.VMEM((S,), jnp.float32),
        pltpu.VMEM((SEG_PER_TILE, D), jnp.float32),
        pltpu.VMEM_SHARED((S, D), jnp.float32),
        pltpu.SemaphoreType.DMA,
        pltpu.SemaphoreType.DMA,
        pltpu.SemaphoreType.DMA,
        pltpu.SemaphoreType.DMA,
        pltpu.SemaphoreType.DMA,
        pltpu.SemaphoreType.DMA,
        pltpu.SemaphoreType.DMA,
        pltpu.SemaphoreType.DMA,
        pltpu.SemaphoreType.DMA,
        pltpu.SemaphoreType.DMA,
        pltpu.SemaphoreType.DMA,
        pltpu.SemaphoreType.DMA,
    ],
)
def _segsum_sc(h_hbm, ids_hbm, psum_hbm, pcnt_hbm,
               id0_v, id1_v, id2_v, id3_v, rows0_v, rows1_v, rows2_v, rows3_v,
               idt_v, rowt_v, hist_v, zsum_v,
               sums_sh, si0, si1, si2, si3, sr0, sr1, sr2, sr3,
               ss0, ss1, ss2, ss3):
    cid = lax.axis_index("c")
    sid = lax.axis_index("s")
    wid = cid * NS + sid
    start_w = wid * KPW + jnp.minimum(wid, REM)
    cnt_w = jnp.where(wid < REM, KPW + 1, KPW)

    zero16 = jnp.zeros((L,), jnp.float32)
    one16 = jnp.ones((L,), jnp.float32)

    NB = 4
    ids_v = (id0_v, id1_v, id2_v, id3_v)
    rows_v = (rows0_v, rows1_v, rows2_v, rows3_v)
    sem_id = (si0, si1, si2, si3)
    sem_in = (sr0, sr1, sr2, sr3)
    sem_sc = (ss0, ss1, ss2, ss3)

    def _id_desc(k):
        return pltpu.make_async_copy(
            ids_hbm.at[pl.ds((start_w + k) * BLK, BLK)], ids_v[k % NB].at[0],
            sem_id[k % NB])

    def _in_desc(k):
        return pltpu.make_async_copy(
            h_hbm.at[pl.ds((start_w + k) * BLK, BLK)], rows_v[k % NB],
            sem_in[k % NB])

    def _sc_desc(k):
        return pltpu.make_async_copy(
            rows_v[k % NB], sums_sh.at[ids_v[k % NB].at[0]], sem_sc[k % NB])

    def start_in(k):
        @pl.when(k < cnt_w)
        def _():
            _id_desc(k).start()
            _in_desc(k).start()

    def wait_sc(k):
        @pl.when(k < cnt_w)
        def _():
            _sc_desc(k).wait()

    start_in(0)
    start_in(1)
    start_in(2)

    @pl.loop(0, S, step=L)
    def _(i):
        hist_v[pl.ds(i, L)] = zero16

    @pl.loop(0, SEG_PER_TILE)
    def _(i):
        @pl.loop(0, D, step=L)
        def _(j):
            zsum_v[i, pl.ds(j, L)] = zero16

    pltpu.sync_copy(zsum_v, sums_sh.at[pl.ds(sid * SEG_PER_TILE, SEG_PER_TILE)])

    plsc.subcore_barrier()

    for k in range(KMAX):
        if k + 3 < KMAX:
            if k >= 1:
                wait_sc(k - 1)
            start_in(k + 3)

        @pl.when(k < cnt_w)
        def _(k=k):
            _id_desc(k).wait()
            _in_desc(k).wait()
            pltpu.async_copy(rows_v[k % NB], sums_sh.at[ids_v[k % NB].at[0]],
                             sem_sc[k % NB], add=True)

            @pl.loop(0, BLK, step=L)
            def _(l):
                idx = ids_v[k % NB][0, pl.ds(l, L)]
                plsc.addupdate_scatter(hist_v, [idx], one16)
    wait_sc(KMAX - 4)
    wait_sc(KMAX - 3)
    wait_sc(KMAX - 2)
    wait_sc(KMAX - 1)

    @pl.when(wid == NW - 1)
    def _():
        base = NFULL * BLK
        pltpu.sync_copy(ids_hbm.at[pl.ds(base, TAIL)], idt_v.at[0])
        pltpu.sync_copy(h_hbm.at[pl.ds(base, TAIL)], rowt_v)
        pltpu.sync_copy(rowt_v, sums_sh.at[idt_v.at[0]], add=True)

        @pl.loop(0, TAIL, step=L)
        def _(l):
            idx = idt_v[0, pl.ds(l, L)]
            plsc.addupdate_scatter(hist_v, [idx], one16)

    plsc.subcore_barrier()

    lo = sid * SEG_PER_TILE
    pltpu.sync_copy(sums_sh.at[pl.ds(lo, SEG_PER_TILE)],
                    psum_hbm.at[cid, pl.ds(lo, SEG_PER_TILE)])
    pltpu.sync_copy(hist_v, pcnt_hbm.at[wid])


def _combine_tc(psum_ref, pcnt_ref, out_ref):
    sums = psum_ref[0] + psum_ref[1]
    cnt = jnp.maximum(jnp.sum(pcnt_ref[...], axis=0), 1.0)
    out_ref[...] = sums / cnt[:, None]


@jax.jit
def kernel(h, segment_ids):
    ids = segment_ids.astype(jnp.int32)
    psum, pcnt = _segsum_sc(h, ids)
    return pl.pallas_call(
        _combine_tc,
        out_shape=jax.ShapeDtypeStruct((S, D), jnp.float32),
    )(psum, pcnt)
